# baseline (speedup 1.0000x reference)
import functools

import jax
import jax.numpy as jnp
from jax import lax
from jax.experimental import pallas as pl
from jax.experimental.pallas import tpu as pltpu
from jax.experimental.pallas import tpu_sc as plsc

_BB = 64

_NC = 2
_NS = 16
_NW = _NC * _NS
_L = 16


def _caps_body(x_ref, dist_ref, feat_ref):
    xb = x_ref[...]
    BB, C, D = xb.shape
    sumsq = jnp.sum(xb * xb, axis=1)
    norm = jnp.sqrt(sumsq)
    mx = jnp.max(norm, axis=1, keepdims=True)
    e = jnp.exp(norm - mx)
    dist_ref[...] = e / jnp.sum(e, axis=1, keepdims=True)
    d_iota = jax.lax.broadcasted_iota(jnp.int32, norm.shape, 1)
    idx = jnp.min(jnp.where(norm == mx, d_iota, D), axis=1,
                  keepdims=True)
    mask = d_iota == idx
    masked = jnp.where(mask[:, None, :], xb, 0.0)
    feat_ref[...] = masked.reshape(BB, C * D)


def _tc_kernel(x):
    B, C, D = x.shape
    return pl.pallas_call(
        _caps_body,
        grid=(B // _BB,),
        in_specs=[pl.BlockSpec((_BB, C, D), lambda i: (i, 0, 0))],
        out_specs=[
            pl.BlockSpec((_BB, D), lambda i: (i, 0)),
            pl.BlockSpec((_BB, C * D), lambda i: (i, 0)),
        ],
        out_shape=[
            jax.ShapeDtypeStruct((B, D), x.dtype),
            jax.ShapeDtypeStruct((B, C * D), x.dtype),
        ],
    )(x)


def _sqrt16(s):
    i = lax.bitcast_convert_type(s, jnp.int32)
    r = lax.bitcast_convert_type(jnp.int32(0x5F3759DF) - (i >> 1), jnp.float32)
    for _ in range(3):
        r = r * (1.5 - 0.5 * s * r * r)
    return s * r


def _rot16(v, lane, sh):
    idx = (lane + sh) & (_L - 1)
    dnums = lax.GatherDimensionNumbers(
        offset_dims=(), collapsed_slice_dims=(0,), start_index_map=(0,))
    return lax.gather(v, idx[:, None], dnums, slice_sizes=(1,),
                      mode=lax.GatherScatterMode.PROMISE_IN_BOUNDS)


def _alltree16(v, lane, op):
    for sh in (8, 4, 2, 1):
        v = op(v, _rot16(v, lane, sh))
    return v


def _sc_kernel(x):
    B, C, D = x.shape
    CD = C * D
    spw = B // _NW
    nch = D // _L

    mesh = plsc.VectorSubcoreMesh(core_axis_name="c", subcore_axis_name="s")

    @functools.partial(
        pl.kernel,
        mesh=mesh,
        out_type=[
            jax.ShapeDtypeStruct((B, D), jnp.float32),
            jax.ShapeDtypeStruct((B, CD), jnp.float32),
        ],
        scratch_types=[
            pltpu.VMEM((C, D), jnp.float32),
            pltpu.VMEM((CD,), jnp.float32),
            pltpu.VMEM((D,), jnp.float32),
        ],
    )
    def sc_caps(x_hbm, dist_hbm, feat_hbm, x_v, f_v, d_v):
        wid = lax.axis_index("s") * _NC + lax.axis_index("c")
        lane = lax.iota(jnp.int32, _L)
        zero16 = jnp.zeros((_L,), jnp.float32)

        def zinit(j, _):
            f_v[pl.ds(j * _L, _L)] = zero16
            return 0
        lax.fori_loop(0, CD // _L, zinit, 0)

        def sample(t, _):
            b = wid * spw + t
            pltpu.sync_copy(x_hbm.at[b], x_v)

            def csum(c, accs):
                out = []
                for k in range(nch):
                    v = x_v[c, pl.ds(k * _L, _L)]
                    out.append(accs[k] + v * v)
                return tuple(out)
            accs = lax.fori_loop(
                0, C, csum, tuple(zero16 for _ in range(nch)))

            m16 = accs[0]
            for k in range(1, nch):
                m16 = jnp.maximum(m16, accs[k])
            smax = _alltree16(m16, lane, jnp.maximum)

            big = jnp.full((_L,), D, jnp.int32)
            i16 = big
            for k in range(nch):
                cand = jnp.where(accs[k] == smax, lane + k * _L, big)
                i16 = jnp.minimum(i16, cand)
            idx = _alltree16(i16, lane, jnp.minimum)
            idx_s = idx[0]
            base = (idx_s // _L) * _L
            off16 = idx - base

            norms = [_sqrt16(accs[k]) for k in range(nch)]
            nmax16 = norms[0]
            for k in range(1, nch):
                nmax16 = jnp.maximum(nmax16, norms[k])
            nmax = _alltree16(nmax16, lane, jnp.maximum)
            es = [jnp.exp(norms[k] - nmax) for k in range(nch)]
            s16 = es[0]
            for k in range(1, nch):
                s16 = s16 + es[k]
            inv = 1.0 / _alltree16(s16, lane, jnp.add)
            for k in range(nch):
                d_v[pl.ds(k * _L, _L)] = es[k] * inv
            pltpu.sync_copy(d_v, dist_hbm.at[b])

            def col_in(c, _):
                w = x_v[c, pl.ds(base, _L)]
                f_v[pl.ds(c * D + base, _L)] = jnp.where(lane == off16, w,
                                                         zero16)
                return 0
            lax.fori_loop(0, C, col_in, 0)
            pltpu.sync_copy(f_v, feat_hbm.at[b])

            def col_out(c, _):
                f_v[pl.ds(c * D + base, _L)] = zero16
                return 0
            lax.fori_loop(0, C, col_out, 0)
            return 0

        lax.fori_loop(0, spw, sample, 0)

    return tuple(sc_caps(x))


def kernel(x):
    return _sc_kernel(x)


# speedup vs baseline: 1.6686x; 1.6686x over previous
import functools

import jax
import jax.numpy as jnp
from jax import lax
from jax.experimental import pallas as pl
from jax.experimental.pallas import tpu as pltpu
from jax.experimental.pallas import tpu_sc as plsc

_BB = 64

_NC = 2
_NS = 16
_NW = _NC * _NS
_L = 16


def _caps_body(x_ref, dist_ref, feat_ref):
    xb = x_ref[...]
    BB, C, D = xb.shape
    sumsq = jnp.sum(xb * xb, axis=1)
    norm = jnp.sqrt(sumsq)
    mx = jnp.max(norm, axis=1, keepdims=True)
    e = jnp.exp(norm - mx)
    dist_ref[...] = e / jnp.sum(e, axis=1, keepdims=True)
    d_iota = jax.lax.broadcasted_iota(jnp.int32, norm.shape, 1)
    idx = jnp.min(jnp.where(norm == mx, d_iota, D), axis=1,
                  keepdims=True)
    mask = d_iota == idx
    masked = jnp.where(mask[:, None, :], xb, 0.0)
    feat_ref[...] = masked.reshape(BB, C * D)


def _tc_kernel(x):
    B, C, D = x.shape
    return pl.pallas_call(
        _caps_body,
        grid=(B // _BB,),
        in_specs=[pl.BlockSpec((_BB, C, D), lambda i: (i, 0, 0))],
        out_specs=[
            pl.BlockSpec((_BB, D), lambda i: (i, 0)),
            pl.BlockSpec((_BB, C * D), lambda i: (i, 0)),
        ],
        out_shape=[
            jax.ShapeDtypeStruct((B, D), x.dtype),
            jax.ShapeDtypeStruct((B, C * D), x.dtype),
        ],
    )(x)


def _sqrt16(s):
    i = lax.bitcast_convert_type(s, jnp.int32)
    r = lax.bitcast_convert_type(jnp.int32(0x5F3759DF) - (i >> 1), jnp.float32)
    for _ in range(3):
        r = r * (1.5 - 0.5 * s * r * r)
    return s * r


def _rot16(v, lane, sh):
    idx = (lane + sh) & (_L - 1)
    dnums = lax.GatherDimensionNumbers(
        offset_dims=(), collapsed_slice_dims=(0,), start_index_map=(0,))
    return lax.gather(v, idx[:, None], dnums, slice_sizes=(1,),
                      mode=lax.GatherScatterMode.PROMISE_IN_BOUNDS)


def _alltree16(v, lane, op):
    for sh in (8, 4, 2, 1):
        v = op(v, _rot16(v, lane, sh))
    return v


def _sc_kernel(x):
    B, C, D = x.shape
    CD = C * D
    spw = B // _NW
    nch = D // _L

    mesh = plsc.VectorSubcoreMesh(core_axis_name="c", subcore_axis_name="s")

    @functools.partial(
        pl.kernel,
        mesh=mesh,
        out_type=[
            jax.ShapeDtypeStruct((B, D), jnp.float32),
            jax.ShapeDtypeStruct((B, CD), jnp.float32),
        ],
        scratch_types=[
            pltpu.VMEM((C, D), jnp.float32),
            pltpu.VMEM((C, D), jnp.float32),
            pltpu.VMEM((CD,), jnp.float32),
            pltpu.VMEM((D,), jnp.float32),
            pltpu.SemaphoreType.DMA,
            pltpu.SemaphoreType.DMA,
            pltpu.SemaphoreType.DMA,
        ],
    )
    def sc_caps(x_hbm, dist_hbm, feat_hbm, x_v0, x_v1, f_v, d_v,
                si0, si1, so):
        wid = lax.axis_index("s") * _NC + lax.axis_index("c")
        b0 = wid * spw
        lane = lax.iota(jnp.int32, _L)
        zero16 = jnp.zeros((_L,), jnp.float32)

        def zinit(j, _):
            f_v[pl.ds(j * _L, _L)] = zero16
            return 0
        lax.fori_loop(0, CD // _L, zinit, 0)

        def process(t, j, x_v, prev_base):
            def csum(c, accs):
                out = []
                for k in range(nch):
                    v = x_v[c, pl.ds(k * _L, _L)]
                    out.append(accs[k] + v * v)
                return tuple(out)
            accs = lax.fori_loop(
                0, C, csum, tuple(zero16 for _ in range(nch)))

            m16 = accs[0]
            for k in range(1, nch):
                m16 = jnp.maximum(m16, accs[k])
            smax = _alltree16(m16, lane, jnp.maximum)

            big = jnp.full((_L,), D, jnp.int32)
            i16 = big
            for k in range(nch):
                cand = jnp.where(accs[k] == smax, lane + k * _L, big)
                i16 = jnp.minimum(i16, cand)
            idx = _alltree16(i16, lane, jnp.minimum)
            idx_s = idx[0]
            base = (idx_s // _L) * _L
            off16 = idx - base

            norms = [_sqrt16(accs[k]) for k in range(nch)]
            nmax16 = norms[0]
            for k in range(1, nch):
                nmax16 = jnp.maximum(nmax16, norms[k])
            nmax = _alltree16(nmax16, lane, jnp.maximum)
            es = [jnp.exp(norms[k] - nmax) for k in range(nch)]
            s16 = es[0]
            for k in range(1, nch):
                s16 = s16 + es[k]
            inv = 1.0 / _alltree16(s16, lane, jnp.add)
            for k in range(nch):
                d_v[pl.ds(k * _L, _L)] = es[k] * inv
            pltpu.sync_copy(d_v, dist_hbm.at[b0 + t])

            @pl.when(t > 0)
            def _():
                pltpu.make_async_copy(f_v, feat_hbm.at[b0], so).wait()

            def col_clear(c, _):
                f_v[pl.ds(c * D + prev_base, _L)] = zero16
                return 0
            lax.fori_loop(0, C, col_clear, 0)

            def col_in(c, _):
                w = x_v[c, pl.ds(base, _L)]
                f_v[pl.ds(c * D + base, _L)] = jnp.where(lane == off16, w,
                                                         zero16)
                return 0
            lax.fori_loop(0, C, col_in, 0)
            pltpu.async_copy(f_v, feat_hbm.at[b0 + t], so)
            return base

        pltpu.async_copy(x_hbm.at[b0], x_v0, si0)

        def body2(j, carry):
            pb0, pb1 = carry
            t0 = 2 * j
            pltpu.async_copy(x_hbm.at[b0 + t0 + 1], x_v1, si1)
            pltpu.make_async_copy(x_hbm.at[b0], x_v0, si0).wait()
            nb0 = process(t0, j, x_v0, pb0)

            @pl.when(j < spw // 2 - 1)
            def _():
                pltpu.async_copy(x_hbm.at[b0 + t0 + 2], x_v0, si0)
            pltpu.make_async_copy(x_hbm.at[b0], x_v1, si1).wait()
            nb1 = process(t0 + 1, j, x_v1, nb0)
            return nb1, nb1

        lax.fori_loop(0, spw // 2, body2, (jnp.int32(0), jnp.int32(0)))
        pltpu.make_async_copy(f_v, feat_hbm.at[b0], so).wait()

    return tuple(sc_caps(x))


def kernel(x):
    return _sc_kernel(x)
